# baseline (speedup 1.0000x reference)
import functools

import jax
import jax.numpy as jnp
from jax import lax
from jax.experimental import pallas as pl
from jax.experimental.pallas import tpu as pltpu
from jax.experimental.pallas import tpu_sc as plsc

N = 10000
E = 320000
D = 128
C = 64

NC = 2
NS = 16
NW = NC * NS
N_PAD = 10240
ROWS_PER_TILE = N_PAD // NS
K = 80
NCH = E // (K * NW)
NBUF = 4


@functools.cache
def _mesh():
    return plsc.VectorSubcoreMesh(
        core_axis_name="c", subcore_axis_name="s", num_cores=NC, num_subcores=NS
    )


def _deg_body(dst_hbm, out_hbm, didx, ones, zbuf, acc, sem_i, sem_s):
    cid = lax.axis_index("c")
    sid = lax.axis_index("s")
    wid = cid * NS + sid

    def fill_ones(i, _):
        ones[pl.ds(i * 16, 16)] = jnp.full((16,), 1.0, jnp.float32)
        return 0

    def fill_zero(i, _):
        zbuf[pl.ds(i * 16, 16)] = jnp.zeros((16,), jnp.float32)
        return 0

    lax.fori_loop(0, K // 16, fill_ones, 0)
    lax.fori_loop(0, ROWS_PER_TILE // 16, fill_zero, 0)
    pltpu.sync_copy(zbuf, acc.at[pl.ds(sid * ROWS_PER_TILE, ROWS_PER_TILE)])
    plsc.subcore_barrier()

    def idx_start(i, b):
        off = (wid + i * NW) * K
        pltpu.async_copy(dst_hbm.at[pl.ds(off, K)], didx.at[b], sem_i.at[b])

    for j in range(2):
        idx_start(j, j)

    def body(i, _):
        b = lax.rem(i, NBUF)
        off = (wid + i * NW) * K
        pltpu.make_async_copy(dst_hbm.at[pl.ds(off, K)], didx.at[b], sem_i.at[b]).wait()

        @pl.when(i >= 2)
        def _():
            b2 = lax.rem(i - 2, NBUF)
            pltpu.make_async_copy(ones, acc.at[didx.at[b2]], sem_s.at[b2]).wait()

        @pl.when(i + 2 < NCH)
        def _():
            idx_start(i + 2, lax.rem(i + 2, NBUF))

        pltpu.async_copy(ones, acc.at[didx.at[b]], sem_s.at[b], add=True)
        return 0

    lax.fori_loop(0, NCH, body, 0)
    for j in (NCH - 2, NCH - 1):
        b = j % NBUF
        pltpu.make_async_copy(ones, acc.at[didx.at[b]], sem_s.at[b]).wait()
    plsc.subcore_barrier()

    pltpu.sync_copy(
        acc.at[pl.ds(sid * ROWS_PER_TILE, ROWS_PER_TILE)],
        out_hbm.at[cid, pl.ds(sid * ROWS_PER_TILE, ROWS_PER_TILE)],
    )


@functools.cache
def _deg_kernel():
    return pl.kernel(
        _deg_body,
        out_type=jax.ShapeDtypeStruct((NC, N_PAD), jnp.float32),
        mesh=_mesh(),
        scratch_types=[
            pltpu.VMEM((NBUF, K), jnp.int32),
            pltpu.VMEM((K,), jnp.float32),
            pltpu.VMEM((ROWS_PER_TILE,), jnp.float32),
            pltpu.VMEM_SHARED((N_PAD,), jnp.float32),
            pltpu.SemaphoreType.DMA((NBUF,)),
            pltpu.SemaphoreType.DMA((NBUF,)),
        ],
    )


def _seg_body(
    x_hbm, src_hbm, dst_hbm, out_hbm,
    sidx, didx, rows, zbuf, acc, sem_si, sem_di, sem_g, sem_s,
):
    cid = lax.axis_index("c")
    sid = lax.axis_index("s")
    wid = cid * NS + sid

    def fill(r, _):
        for c8 in range(D // 16):
            zbuf[r, pl.ds(c8 * 16, 16)] = jnp.zeros((16,), jnp.float32)
        return 0

    lax.fori_loop(0, 32, fill, 0)

    def zero_out(t, _):
        pltpu.sync_copy(zbuf, acc.at[pl.ds(sid * ROWS_PER_TILE + t * 32, 32)])
        return 0

    lax.fori_loop(0, ROWS_PER_TILE // 32, zero_out, 0)
    plsc.subcore_barrier()

    def idx_start(i, b):
        off = (wid + i * NW) * K
        pltpu.async_copy(src_hbm.at[pl.ds(off, K)], sidx.at[b], sem_si.at[b])
        pltpu.async_copy(dst_hbm.at[pl.ds(off, K)], didx.at[b], sem_di.at[b])

    for j in range(2):
        idx_start(j, j)

    def body(i, _):
        b = lax.rem(i, NBUF)
        off = (wid + i * NW) * K
        pltpu.make_async_copy(src_hbm.at[pl.ds(off, K)], sidx.at[b], sem_si.at[b]).wait()
        pltpu.make_async_copy(dst_hbm.at[pl.ds(off, K)], didx.at[b], sem_di.at[b]).wait()
        pltpu.async_copy(x_hbm.at[sidx.at[b]], rows.at[b], sem_g.at[b])

        @pl.when(i >= 2)
        def _():
            b2 = lax.rem(i - 2, NBUF)
            pltpu.make_async_copy(rows.at[b2], acc.at[didx.at[b2]], sem_s.at[b2]).wait()

        @pl.when(i + 2 < NCH)
        def _():
            idx_start(i + 2, lax.rem(i + 2, NBUF))

        pltpu.make_async_copy(x_hbm.at[sidx.at[b]], rows.at[b], sem_g.at[b]).wait()
        pltpu.async_copy(rows.at[b], acc.at[didx.at[b]], sem_s.at[b], add=True)
        return 0

    lax.fori_loop(0, NCH, body, 0)
    for j in (NCH - 2, NCH - 1):
        b = j % NBUF
        pltpu.make_async_copy(rows.at[b], acc.at[didx.at[b]], sem_s.at[b]).wait()
    plsc.subcore_barrier()

    pltpu.sync_copy(
        acc.at[pl.ds(sid * ROWS_PER_TILE, ROWS_PER_TILE)],
        out_hbm.at[cid, pl.ds(sid * ROWS_PER_TILE, ROWS_PER_TILE)],
    )


@functools.cache
def _seg_kernel():
    return pl.kernel(
        _seg_body,
        out_type=jax.ShapeDtypeStruct((NC, N_PAD, D), jnp.float32),
        mesh=_mesh(),
        scratch_types=[
            pltpu.VMEM((NBUF, K), jnp.int32),
            pltpu.VMEM((NBUF, K), jnp.int32),
            pltpu.VMEM((NBUF, K, D), jnp.float32),
            pltpu.VMEM((32, D), jnp.float32),
            pltpu.VMEM_SHARED((N_PAD, D), jnp.float32),
            pltpu.SemaphoreType.DMA((NBUF,)),
            pltpu.SemaphoreType.DMA((NBUF,)),
            pltpu.SemaphoreType.DMA((NBUF,)),
            pltpu.SemaphoreType.DMA((NBUF,)),
        ],
    )


def _norm_scale_body(degp_ref, x_ref, a_ref, nrm_ref):
    deg = degp_ref[0, :N] + degp_ref[1, :N]
    nrm = lax.rsqrt(jnp.maximum(deg, 1.0))
    nrm_ref[...] = nrm
    a_ref[...] = x_ref[...] * nrm[:, None]


def _combine_scale_body(part_ref, nrm_ref, c_ref):
    nrm = nrm_ref[...]
    c_ref[...] = (part_ref[0, :N] + part_ref[1, :N]) * (nrm * nrm)[:, None]


def _final_body(part_ref, nrm_ref, w_ref, out_ref, h_ref):
    h2 = (part_ref[0, :N] + part_ref[1, :N]) * nrm_ref[...][:, None]
    h = jnp.dot(h2, w_ref[...], preferred_element_type=jnp.float32)
    h_ref[...] = h
    m = jnp.max(h, axis=1, keepdims=True)
    e = jnp.exp(h - m)
    out_ref[...] = e / jnp.sum(e, axis=1, keepdims=True)


def kernel(features, edge_index, lg, lg_x, W):
    del lg, lg_x
    src = edge_index[0]
    dst = edge_index[1]

    degp = _deg_kernel()(dst)

    a, nrm = pl.pallas_call(
        _norm_scale_body,
        out_shape=(
            jax.ShapeDtypeStruct((N, D), jnp.float32),
            jax.ShapeDtypeStruct((N,), jnp.float32),
        ),
    )(degp, features)

    bp = _seg_kernel()(a, src, dst)

    c = pl.pallas_call(
        _combine_scale_body,
        out_shape=jax.ShapeDtypeStruct((N, D), jnp.float32),
    )(bp, nrm)

    dp = _seg_kernel()(c, src, dst)

    out, h = pl.pallas_call(
        _final_body,
        out_shape=(
            jax.ShapeDtypeStruct((N, C), jnp.float32),
            jax.ShapeDtypeStruct((N, C), jnp.float32),
        ),
    )(dp, nrm, W)

    return (out, h)


# speedup vs baseline: 1.1849x; 1.1849x over previous
import functools

import jax
import jax.numpy as jnp
from jax import lax
from jax.experimental import pallas as pl
from jax.experimental.pallas import tpu as pltpu
from jax.experimental.pallas import tpu_sc as plsc

N = 10000
E = 320000
D = 128
C = 64

NC = 2
NS = 16
NW = NC * NS
N_PAD = 10240
ROWS_PER_TILE = N_PAD // NS
K = 128
NCHUNKS = E // K
NBUF = 4


def _n_my_chunks(wid):
    return jnp.where(wid < NCHUNKS % NW, NCHUNKS // NW + 1, NCHUNKS // NW)


@functools.cache
def _mesh():
    return plsc.VectorSubcoreMesh(
        core_axis_name="c", subcore_axis_name="s", num_cores=NC, num_subcores=NS
    )


def _deg_body(dst_hbm, out_hbm, didx, ones, zbuf, acc, sem_i, sem_s):
    cid = lax.axis_index("c")
    sid = lax.axis_index("s")
    wid = cid * NS + sid

    def fill_ones(i, _):
        ones[pl.ds(i * 16, 16)] = jnp.full((16,), 1.0, jnp.float32)
        return 0

    def fill_zero(i, _):
        zbuf[pl.ds(i * 16, 16)] = jnp.zeros((16,), jnp.float32)
        return 0

    lax.fori_loop(0, K // 16, fill_ones, 0)
    lax.fori_loop(0, ROWS_PER_TILE // 16, fill_zero, 0)
    pltpu.sync_copy(zbuf, acc.at[pl.ds(sid * ROWS_PER_TILE, ROWS_PER_TILE)])
    plsc.subcore_barrier()

    nch = _n_my_chunks(wid)

    def idx_start(i, b):
        off = (wid + i * NW) * K
        pltpu.async_copy(dst_hbm.at[pl.ds(off, K)], didx.at[b], sem_i.at[b])

    for j in range(2):
        idx_start(j, j)

    def body(i, _):
        b = lax.rem(i, NBUF)
        off = (wid + i * NW) * K
        pltpu.make_async_copy(dst_hbm.at[pl.ds(off, K)], didx.at[b], sem_i.at[b]).wait()

        @pl.when(i >= 2)
        def _():
            b2 = lax.rem(i - 2, NBUF)
            pltpu.make_async_copy(ones, acc.at[didx.at[b2]], sem_s.at[b2]).wait()

        @pl.when(i + 2 < nch)
        def _():
            idx_start(i + 2, lax.rem(i + 2, NBUF))

        pltpu.async_copy(ones, acc.at[didx.at[b]], sem_s.at[b], add=True)
        return 0

    lax.fori_loop(0, nch, body, 0)
    for dj in (2, 1):
        b = lax.rem(nch - dj, NBUF)
        pltpu.make_async_copy(ones, acc.at[didx.at[b]], sem_s.at[b]).wait()
    plsc.subcore_barrier()

    pltpu.sync_copy(
        acc.at[pl.ds(sid * ROWS_PER_TILE, ROWS_PER_TILE)],
        out_hbm.at[cid, pl.ds(sid * ROWS_PER_TILE, ROWS_PER_TILE)],
    )


@functools.cache
def _deg_kernel():
    return pl.kernel(
        _deg_body,
        out_type=jax.ShapeDtypeStruct((NC, N_PAD), jnp.float32),
        mesh=_mesh(),
        scratch_types=[
            pltpu.VMEM((NBUF, K), jnp.int32),
            pltpu.VMEM((K,), jnp.float32),
            pltpu.VMEM((ROWS_PER_TILE,), jnp.float32),
            pltpu.VMEM_SHARED((N_PAD,), jnp.float32),
            pltpu.SemaphoreType.DMA((NBUF,)),
            pltpu.SemaphoreType.DMA((NBUF,)),
        ],
    )


def _seg_body(
    x_hbm, src_hbm, dst_hbm, out_hbm,
    sidx, didx, rows, zbuf, acc, sem_si, sem_di, sem_g, sem_s,
):
    cid = lax.axis_index("c")
    sid = lax.axis_index("s")
    wid = cid * NS + sid

    def fill(r, _):
        for c8 in range(D // 16):
            zbuf[r, pl.ds(c8 * 16, 16)] = jnp.zeros((16,), jnp.float32)
        return 0

    lax.fori_loop(0, 32, fill, 0)

    def zero_out(t, _):
        pltpu.sync_copy(zbuf, acc.at[pl.ds(sid * ROWS_PER_TILE + t * 32, 32)])
        return 0

    lax.fori_loop(0, ROWS_PER_TILE // 32, zero_out, 0)
    plsc.subcore_barrier()

    nch = _n_my_chunks(wid)

    def idx_start(i, b):
        off = (wid + i * NW) * K
        pltpu.async_copy(src_hbm.at[pl.ds(off, K)], sidx.at[b], sem_si.at[b])
        pltpu.async_copy(dst_hbm.at[pl.ds(off, K)], didx.at[b], sem_di.at[b])

    for j in range(2):
        idx_start(j, j)

    def body(i, _):
        b4 = lax.rem(i, NBUF)
        b2 = lax.rem(i, 2)
        off = (wid + i * NW) * K
        pltpu.make_async_copy(src_hbm.at[pl.ds(off, K)], sidx.at[b4], sem_si.at[b4]).wait()
        pltpu.make_async_copy(dst_hbm.at[pl.ds(off, K)], didx.at[b4], sem_di.at[b4]).wait()

        @pl.when(i >= 2)
        def _():
            pltpu.make_async_copy(
                rows.at[b2], acc.at[didx.at[lax.rem(i - 2, NBUF)]],
                sem_s.at[lax.rem(i - 2, NBUF)],
            ).wait()

        pltpu.async_copy(x_hbm.at[sidx.at[b4]], rows.at[b2], sem_g.at[b2])

        @pl.when(i + 2 < nch)
        def _():
            idx_start(i + 2, lax.rem(i + 2, NBUF))

        pltpu.make_async_copy(x_hbm.at[sidx.at[b4]], rows.at[b2], sem_g.at[b2]).wait()
        pltpu.async_copy(rows.at[b2], acc.at[didx.at[b4]], sem_s.at[b4], add=True)
        return 0

    lax.fori_loop(0, nch, body, 0)
    for dj in (2, 1):
        b4 = lax.rem(nch - dj, NBUF)
        b2 = lax.rem(nch - dj, 2)
        pltpu.make_async_copy(rows.at[b2], acc.at[didx.at[b4]], sem_s.at[b4]).wait()
    plsc.subcore_barrier()

    pltpu.sync_copy(
        acc.at[pl.ds(sid * ROWS_PER_TILE, ROWS_PER_TILE)],
        out_hbm.at[cid, pl.ds(sid * ROWS_PER_TILE, ROWS_PER_TILE)],
    )


@functools.cache
def _seg_kernel():
    return pl.kernel(
        _seg_body,
        out_type=jax.ShapeDtypeStruct((NC, N_PAD, D), jnp.float32),
        mesh=_mesh(),
        scratch_types=[
            pltpu.VMEM((NBUF, K), jnp.int32),
            pltpu.VMEM((NBUF, K), jnp.int32),
            pltpu.VMEM((2, K, D), jnp.float32),
            pltpu.VMEM((32, D), jnp.float32),
            pltpu.VMEM_SHARED((N_PAD, D), jnp.float32),
            pltpu.SemaphoreType.DMA((NBUF,)),
            pltpu.SemaphoreType.DMA((NBUF,)),
            pltpu.SemaphoreType.DMA((2,)),
            pltpu.SemaphoreType.DMA((NBUF,)),
        ],
    )


def _norm_scale_body(degp_ref, x_ref, a_ref, nrm_ref):
    deg = degp_ref[0, :N] + degp_ref[1, :N]
    nrm = lax.rsqrt(jnp.maximum(deg, 1.0))
    nrm_ref[...] = nrm
    a_ref[...] = x_ref[...] * nrm[:, None]


def _combine_scale_body(part_ref, nrm_ref, c_ref):
    nrm = nrm_ref[...]
    c_ref[...] = (part_ref[0, :N] + part_ref[1, :N]) * (nrm * nrm)[:, None]


def _final_body(part_ref, nrm_ref, w_ref, out_ref, h_ref):
    h2 = (part_ref[0, :N] + part_ref[1, :N]) * nrm_ref[...][:, None]
    h = jnp.dot(h2, w_ref[...], preferred_element_type=jnp.float32)
    h_ref[...] = h
    m = jnp.max(h, axis=1, keepdims=True)
    e = jnp.exp(h - m)
    out_ref[...] = e / jnp.sum(e, axis=1, keepdims=True)


def kernel(features, edge_index, lg, lg_x, W):
    del lg, lg_x
    src = edge_index[0]
    dst = edge_index[1]

    degp = _deg_kernel()(dst)

    a, nrm = pl.pallas_call(
        _norm_scale_body,
        out_shape=(
            jax.ShapeDtypeStruct((N, D), jnp.float32),
            jax.ShapeDtypeStruct((N,), jnp.float32),
        ),
    )(degp, features)

    bp = _seg_kernel()(a, src, dst)

    c = pl.pallas_call(
        _combine_scale_body,
        out_shape=jax.ShapeDtypeStruct((N, D), jnp.float32),
    )(bp, nrm)

    dp = _seg_kernel()(c, src, dst)

    out, h = pl.pallas_call(
        _final_body,
        out_shape=(
            jax.ShapeDtypeStruct((N, C), jnp.float32),
            jax.ShapeDtypeStruct((N, C), jnp.float32),
        ),
    )(dp, nrm, W)

    return (out, h)


# speedup vs baseline: 1.4517x; 1.2251x over previous
import functools

import jax
import jax.numpy as jnp
from jax import lax
from jax.experimental import pallas as pl
from jax.experimental.pallas import tpu as pltpu
from jax.experimental.pallas import tpu_sc as plsc

N = 10000
E = 320000
D = 128
C = 64

NC = 2
NS = 16
NW = NC * NS
N_PAD = 10112
ROWS_PER_TILE = N_PAD // NS
N_PAD_DEG = 10240
RPT_DEG = N_PAD_DEG // NS
K = 128
NCHUNKS = E // K
NBUF = 4
_SCATTER_ON = True
_GATHER_ON = True


def _n_my_chunks(wid):
    return jnp.where(wid < NCHUNKS % NW, NCHUNKS // NW + 1, NCHUNKS // NW)


@functools.cache
def _mesh():
    return plsc.VectorSubcoreMesh(
        core_axis_name="c", subcore_axis_name="s", num_cores=NC, num_subcores=NS
    )


def _deg_body(dst_hbm, out_hbm, didx, ones, zbuf, acc, sem_i, sem_s):
    cid = lax.axis_index("c")
    sid = lax.axis_index("s")
    wid = cid * NS + sid

    def fill_ones(i, _):
        ones[pl.ds(i * 16, 16)] = jnp.full((16,), 1.0, jnp.float32)
        return 0

    def fill_zero(i, _):
        zbuf[pl.ds(i * 16, 16)] = jnp.zeros((16,), jnp.float32)
        return 0

    lax.fori_loop(0, K // 16, fill_ones, 0)
    lax.fori_loop(0, RPT_DEG // 16, fill_zero, 0)
    pltpu.sync_copy(zbuf, acc.at[pl.ds(sid * RPT_DEG, RPT_DEG)])
    plsc.subcore_barrier()

    nch = _n_my_chunks(wid)

    def idx_start(i, b):
        off = (wid + i * NW) * K
        pltpu.async_copy(dst_hbm.at[pl.ds(off, K)], didx.at[b], sem_i.at[b])

    for j in range(2):
        idx_start(j, j)

    def body(i, _):
        b = lax.rem(i, NBUF)
        off = (wid + i * NW) * K
        pltpu.make_async_copy(dst_hbm.at[pl.ds(off, K)], didx.at[b], sem_i.at[b]).wait()

        @pl.when(i >= 2)
        def _():
            b2 = lax.rem(i - 2, NBUF)
            pltpu.make_async_copy(ones, acc.at[didx.at[b2]], sem_s.at[b2]).wait()

        @pl.when(i + 2 < nch)
        def _():
            idx_start(i + 2, lax.rem(i + 2, NBUF))

        pltpu.async_copy(ones, acc.at[didx.at[b]], sem_s.at[b], add=True)
        return 0

    lax.fori_loop(0, nch, body, 0)
    for dj in (2, 1):
        b = lax.rem(nch - dj, NBUF)
        pltpu.make_async_copy(ones, acc.at[didx.at[b]], sem_s.at[b]).wait()
    plsc.subcore_barrier()

    pltpu.sync_copy(
        acc.at[pl.ds(sid * RPT_DEG, RPT_DEG)],
        out_hbm.at[cid, pl.ds(sid * RPT_DEG, RPT_DEG)],
    )


@functools.cache
def _deg_kernel():
    return pl.kernel(
        _deg_body,
        out_type=jax.ShapeDtypeStruct((NC, N_PAD_DEG), jnp.float32),
        mesh=_mesh(),
        scratch_types=[
            pltpu.VMEM((NBUF, K), jnp.int32),
            pltpu.VMEM((K,), jnp.float32),
            pltpu.VMEM((RPT_DEG,), jnp.float32),
            pltpu.VMEM_SHARED((N_PAD_DEG,), jnp.float32),
            pltpu.SemaphoreType.DMA((NBUF,)),
            pltpu.SemaphoreType.DMA((NBUF,)),
        ],
    )


def _seg_body(
    x_hbm, src_hbm, dst_hbm, out_hbm,
    sidx, didx, rows, acc, sem_si, sem_di, sem_g, sem_s,
):
    cid = lax.axis_index("c")
    sid = lax.axis_index("s")
    wid = cid * NS + sid

    def fill(r, _):
        for c8 in range(D // 16):
            rows[0, r, pl.ds(c8 * 16, 16)] = jnp.zeros((16,), jnp.float32)
        return 0

    lax.fori_loop(0, 79, fill, 0)

    def zero_out(t, _):
        pltpu.sync_copy(
            rows.at[0, pl.ds(0, 79)],
            acc.at[pl.ds(sid * ROWS_PER_TILE + t * 79, 79)],
        )
        return 0

    lax.fori_loop(0, ROWS_PER_TILE // 79, zero_out, 0)
    plsc.subcore_barrier()

    nch = _n_my_chunks(wid)

    def idx_start(i, bs, bd):
        off = (wid + i * NW) * K
        pltpu.async_copy(src_hbm.at[pl.ds(off, K)], sidx.at[bs], sem_si.at[bs])
        pltpu.async_copy(dst_hbm.at[pl.ds(off, K)], didx.at[bd], sem_di.at[bd])

    def idx_wait_s(i, bs):
        off = (wid + i * NW) * K
        pltpu.make_async_copy(src_hbm.at[pl.ds(off, K)], sidx.at[bs], sem_si.at[bs]).wait()

    def gather_start(i, bs, b3):
        pltpu.async_copy(x_hbm.at[sidx.at[bs]], rows.at[b3], sem_g.at[b3])

    idx_start(0, 0, 0)
    idx_start(1, 1, 1)
    idx_wait_s(0, 0)
    gather_start(0, 0, 0)

    def body(i, _):
        b3 = lax.rem(i, 3)
        b4 = lax.rem(i, NBUF)
        off = (wid + i * NW) * K

        @pl.when(i >= 2)
        def _():
            pltpu.make_async_copy(
                rows.at[lax.rem(i - 2, 3)], acc.at[didx.at[lax.rem(i - 2, NBUF)]],
                sem_s.at[lax.rem(i - 2, NBUF)],
            ).wait()

        @pl.when(i + 2 < nch)
        def _():
            idx_start(i + 2, lax.rem(i + 2, 3), lax.rem(i + 2, NBUF))

        @pl.when(i + 1 < nch)
        def _():
            bs1 = lax.rem(i + 1, 3)
            idx_wait_s(i + 1, bs1)
            gather_start(i + 1, bs1, bs1)

        pltpu.make_async_copy(x_hbm.at[sidx.at[b3]], rows.at[b3], sem_g.at[b3]).wait()
        pltpu.make_async_copy(dst_hbm.at[pl.ds(off, K)], didx.at[b4], sem_di.at[b4]).wait()
        pltpu.async_copy(rows.at[b3], acc.at[didx.at[b4]], sem_s.at[b4], add=True)
        return 0

    lax.fori_loop(0, nch, body, 0)
    for dj in (2, 1):
        b4 = lax.rem(nch - dj, NBUF)
        b3 = lax.rem(nch - dj, 3)
        pltpu.make_async_copy(rows.at[b3], acc.at[didx.at[b4]], sem_s.at[b4]).wait()
    plsc.subcore_barrier()

    pltpu.sync_copy(
        acc.at[pl.ds(sid * ROWS_PER_TILE, ROWS_PER_TILE)],
        out_hbm.at[cid, pl.ds(sid * ROWS_PER_TILE, ROWS_PER_TILE)],
    )


@functools.cache
def _seg_kernel():
    return pl.kernel(
        _seg_body,
        out_type=jax.ShapeDtypeStruct((NC, N_PAD, D), jnp.float32),
        mesh=_mesh(),
        scratch_types=[
            pltpu.VMEM((3, K), jnp.int32),
            pltpu.VMEM((NBUF, K), jnp.int32),
            pltpu.VMEM((3, K, D), jnp.float32),
            pltpu.VMEM_SHARED((N_PAD, D), jnp.float32),
            pltpu.SemaphoreType.DMA((3,)),
            pltpu.SemaphoreType.DMA((NBUF,)),
            pltpu.SemaphoreType.DMA((3,)),
            pltpu.SemaphoreType.DMA((NBUF,)),
        ],
    )


def _norm_scale_body(degp_ref, x_ref, a_ref, nrm_ref):
    deg = degp_ref[0, :N] + degp_ref[1, :N]
    nrm = lax.rsqrt(jnp.maximum(deg, 1.0))
    nrm_ref[...] = nrm
    a_ref[...] = x_ref[...] * nrm[:, None]


def _combine_scale_body(part_ref, nrm_ref, c_ref):
    nrm = nrm_ref[...]
    c_ref[...] = (part_ref[0, :N] + part_ref[1, :N]) * (nrm * nrm)[:, None]


def _final_body(part_ref, nrm_ref, w_ref, out_ref, h_ref):
    h2 = (part_ref[0, :N] + part_ref[1, :N]) * nrm_ref[...][:, None]
    h = jnp.dot(h2, w_ref[...], preferred_element_type=jnp.float32)
    h_ref[...] = h
    m = jnp.max(h, axis=1, keepdims=True)
    e = jnp.exp(h - m)
    out_ref[...] = e / jnp.sum(e, axis=1, keepdims=True)


def kernel(features, edge_index, lg, lg_x, W):
    del lg, lg_x
    src = edge_index[0]
    dst = edge_index[1]

    degp = _deg_kernel()(dst)

    a, nrm = pl.pallas_call(
        _norm_scale_body,
        out_shape=(
            jax.ShapeDtypeStruct((N, D), jnp.float32),
            jax.ShapeDtypeStruct((N,), jnp.float32),
        ),
    )(degp, features)

    bp = _seg_kernel()(a, src, dst)

    c = pl.pallas_call(
        _combine_scale_body,
        out_shape=jax.ShapeDtypeStruct((N, D), jnp.float32),
    )(bp, nrm)

    dp = _seg_kernel()(c, src, dst)

    out, h = pl.pallas_call(
        _final_body,
        out_shape=(
            jax.ShapeDtypeStruct((N, C), jnp.float32),
            jax.ShapeDtypeStruct((N, C), jnp.float32),
        ),
    )(dp, nrm, W)

    return (out, h)


# speedup vs baseline: 1.5007x; 1.0338x over previous
import functools

import jax
import jax.numpy as jnp
from jax import lax
from jax.experimental import pallas as pl
from jax.experimental.pallas import tpu as pltpu
from jax.experimental.pallas import tpu_sc as plsc

N = 10000
E = 320000
D = 128
C = 64

NC = 2
NS = 16
NW = NC * NS
N_PAD = 10112
ROWS_PER_TILE = N_PAD // NS
N_PAD_DEG = 10240
RPT_DEG = N_PAD_DEG // NS
K = 128
NCHUNKS = E // K
NBUF = 4
_SCATTER_ON = True
_GATHER_ON = True


def _n_my_chunks(wid):
    return jnp.where(wid < NCHUNKS % NW, NCHUNKS // NW + 1, NCHUNKS // NW)


@functools.cache
def _mesh():
    return plsc.VectorSubcoreMesh(
        core_axis_name="c", subcore_axis_name="s", num_cores=NC, num_subcores=NS
    )


def _deg_body(dst_hbm, out_hbm, didx, ones, zbuf, acc, sem_i, sem_s):
    cid = lax.axis_index("c")
    sid = lax.axis_index("s")
    wid = cid * NS + sid

    def fill_ones(i, _):
        ones[pl.ds(i * 16, 16)] = jnp.full((16,), 1.0, jnp.float32)
        return 0

    def fill_zero(i, _):
        zbuf[pl.ds(i * 16, 16)] = jnp.zeros((16,), jnp.float32)
        return 0

    lax.fori_loop(0, K // 16, fill_ones, 0)
    lax.fori_loop(0, RPT_DEG // 16, fill_zero, 0)
    pltpu.sync_copy(zbuf, acc.at[pl.ds(sid * RPT_DEG, RPT_DEG)])
    plsc.subcore_barrier()

    nch = _n_my_chunks(wid)

    def idx_start(i, b):
        off = (wid + i * NW) * K
        pltpu.async_copy(dst_hbm.at[pl.ds(off, K)], didx.at[b], sem_i.at[b])

    for j in range(4):
        idx_start(j, j)

    def body(i, _):
        b = lax.rem(i, 8)
        off = (wid + i * NW) * K
        pltpu.make_async_copy(dst_hbm.at[pl.ds(off, K)], didx.at[b], sem_i.at[b]).wait()

        @pl.when(i >= 4)
        def _():
            b2 = lax.rem(i - 4, 8)
            pltpu.make_async_copy(ones, acc.at[didx.at[b2]], sem_s.at[b2]).wait()

        @pl.when(i + 4 < nch)
        def _():
            idx_start(i + 4, lax.rem(i + 4, 8))

        pltpu.async_copy(ones, acc.at[didx.at[b]], sem_s.at[b], add=True)
        return 0

    lax.fori_loop(0, nch, body, 0)
    for dj in (4, 3, 2, 1):
        b = lax.rem(nch - dj, 8)
        pltpu.make_async_copy(ones, acc.at[didx.at[b]], sem_s.at[b]).wait()
    plsc.subcore_barrier()

    pltpu.sync_copy(
        acc.at[pl.ds(sid * RPT_DEG, RPT_DEG)],
        out_hbm.at[cid, pl.ds(sid * RPT_DEG, RPT_DEG)],
    )


@functools.cache
def _deg_kernel():
    return pl.kernel(
        _deg_body,
        out_type=jax.ShapeDtypeStruct((NC, N_PAD_DEG), jnp.float32),
        mesh=_mesh(),
        scratch_types=[
            pltpu.VMEM((8, K), jnp.int32),
            pltpu.VMEM((K,), jnp.float32),
            pltpu.VMEM((RPT_DEG,), jnp.float32),
            pltpu.VMEM_SHARED((N_PAD_DEG,), jnp.float32),
            pltpu.SemaphoreType.DMA((8,)),
            pltpu.SemaphoreType.DMA((8,)),
        ],
    )


def _seg_body(
    x_hbm, src_hbm, dst_hbm, out_hbm,
    sidx, didx, rows, acc, sem_si, sem_di, sem_g, sem_s,
):
    cid = lax.axis_index("c")
    sid = lax.axis_index("s")
    wid = cid * NS + sid

    def fill(r, _):
        for c8 in range(D // 16):
            rows[0, r, pl.ds(c8 * 16, 16)] = jnp.zeros((16,), jnp.float32)
        return 0

    lax.fori_loop(0, 79, fill, 0)

    def zero_out(t, _):
        pltpu.sync_copy(
            rows.at[0, pl.ds(0, 79)],
            acc.at[pl.ds(sid * ROWS_PER_TILE + t * 79, 79)],
        )
        return 0

    lax.fori_loop(0, ROWS_PER_TILE // 79, zero_out, 0)
    plsc.subcore_barrier()

    nch = _n_my_chunks(wid)

    def idx_start(i, bs, bd):
        off = (wid + i * NW) * K
        pltpu.async_copy(src_hbm.at[pl.ds(off, K)], sidx.at[bs], sem_si.at[bs])
        pltpu.async_copy(dst_hbm.at[pl.ds(off, K)], didx.at[bd], sem_di.at[bd])

    def idx_wait_s(i, bs):
        off = (wid + i * NW) * K
        pltpu.make_async_copy(src_hbm.at[pl.ds(off, K)], sidx.at[bs], sem_si.at[bs]).wait()

    def gather_start(i, bs, b3):
        pltpu.async_copy(x_hbm.at[sidx.at[bs]], rows.at[b3], sem_g.at[b3])

    idx_start(0, 0, 0)
    idx_start(1, 1, 1)
    idx_wait_s(0, 0)
    gather_start(0, 0, 0)

    def body(i, _):
        b3 = lax.rem(i, 3)
        b4 = lax.rem(i, NBUF)
        off = (wid + i * NW) * K

        @pl.when(i >= 2)
        def _():
            pltpu.make_async_copy(
                rows.at[lax.rem(i - 2, 3)], acc.at[didx.at[lax.rem(i - 2, NBUF)]],
                sem_s.at[lax.rem(i - 2, NBUF)],
            ).wait()

        @pl.when(i + 2 < nch)
        def _():
            idx_start(i + 2, lax.rem(i + 2, 3), lax.rem(i + 2, NBUF))

        @pl.when(i + 1 < nch)
        def _():
            bs1 = lax.rem(i + 1, 3)
            idx_wait_s(i + 1, bs1)
            gather_start(i + 1, bs1, bs1)

        pltpu.make_async_copy(x_hbm.at[sidx.at[b3]], rows.at[b3], sem_g.at[b3]).wait()
        pltpu.make_async_copy(dst_hbm.at[pl.ds(off, K)], didx.at[b4], sem_di.at[b4]).wait()
        pltpu.async_copy(rows.at[b3], acc.at[didx.at[b4]], sem_s.at[b4], add=True)
        return 0

    lax.fori_loop(0, nch, body, 0)
    for dj in (2, 1):
        b4 = lax.rem(nch - dj, NBUF)
        b3 = lax.rem(nch - dj, 3)
        pltpu.make_async_copy(rows.at[b3], acc.at[didx.at[b4]], sem_s.at[b4]).wait()
    plsc.subcore_barrier()

    pltpu.sync_copy(
        acc.at[pl.ds(sid * ROWS_PER_TILE, ROWS_PER_TILE)],
        out_hbm.at[cid, pl.ds(sid * ROWS_PER_TILE, ROWS_PER_TILE)],
    )


@functools.cache
def _seg_kernel():
    return pl.kernel(
        _seg_body,
        out_type=jax.ShapeDtypeStruct((NC, N_PAD, D), jnp.float32),
        mesh=_mesh(),
        scratch_types=[
            pltpu.VMEM((3, K), jnp.int32),
            pltpu.VMEM((NBUF, K), jnp.int32),
            pltpu.VMEM((3, K, D), jnp.float32),
            pltpu.VMEM_SHARED((N_PAD, D), jnp.float32),
            pltpu.SemaphoreType.DMA((3,)),
            pltpu.SemaphoreType.DMA((NBUF,)),
            pltpu.SemaphoreType.DMA((3,)),
            pltpu.SemaphoreType.DMA((NBUF,)),
        ],
    )


def _norm_scale_body(degp_ref, x_ref, a_ref, nrm_ref):
    deg = degp_ref[0, :N] + degp_ref[1, :N]
    nrm = lax.rsqrt(jnp.maximum(deg, 1.0))
    nrm_ref[...] = nrm
    a_ref[...] = x_ref[...] * nrm[:, None]


def _combine_scale_body(part_ref, nrm_ref, c_ref):
    nrm = nrm_ref[...]
    c_ref[...] = (part_ref[0, :N] + part_ref[1, :N]) * (nrm * nrm)[:, None]


def _final_body(part_ref, nrm_ref, w_ref, out_ref, h_ref):
    h2 = (part_ref[0, :N] + part_ref[1, :N]) * nrm_ref[...][:, None]
    h = jnp.dot(h2, w_ref[...], preferred_element_type=jnp.float32)
    h_ref[...] = h
    m = jnp.max(h, axis=1, keepdims=True)
    e = jnp.exp(h - m)
    out_ref[...] = e / jnp.sum(e, axis=1, keepdims=True)


def kernel(features, edge_index, lg, lg_x, W):
    del lg, lg_x
    src = edge_index[0]
    dst = edge_index[1]

    degp = _deg_kernel()(dst)

    a, nrm = pl.pallas_call(
        _norm_scale_body,
        out_shape=(
            jax.ShapeDtypeStruct((N, D), jnp.float32),
            jax.ShapeDtypeStruct((N,), jnp.float32),
        ),
    )(degp, features)

    bp = _seg_kernel()(a, src, dst)

    c = pl.pallas_call(
        _combine_scale_body,
        out_shape=jax.ShapeDtypeStruct((N, D), jnp.float32),
    )(bp, nrm)

    dp = _seg_kernel()(c, src, dst)

    out, h = pl.pallas_call(
        _final_body,
        out_shape=(
            jax.ShapeDtypeStruct((N, C), jnp.float32),
            jax.ShapeDtypeStruct((N, C), jnp.float32),
        ),
    )(dp, nrm, W)

    return (out, h)
